# baseline (speedup 1.0000x reference)
import functools

import numpy as np
import jax
import jax.numpy as jnp
from jax import lax
from jax.experimental import pallas as pl
from jax.experimental.pallas import tpu as pltpu
from jax.experimental.pallas import tpu_sc as plsc

N_NODES = 10000
N_EDGES = 320000
D = 128

NC = 2
NS = 16
NW = NC * NS

C = 128
CH = 80
PAD_E = NW * CH * C

R_BASE = 624
ZR = 8


def _sc_body(rows_hbm, cols_hbm, vals_hbm, x_hbm, out_hbm,
             cols_v, rows_v, vals_v, gath_v, sb0, acc, semg, sems0):
    c = lax.axis_index("c")
    s = lax.axis_index("s")
    wid = s * NC + c

    zeros16 = jnp.zeros((16,), jnp.float32)
    for r in range(ZR):
        for j in range(D // 16):
            sb0[r, pl.ds(j * 16, 16)] = zeros16

    def zloop(i, carry):
        pltpu.sync_copy(sb0.at[pl.ds(0, ZR)],
                        acc.at[pl.ds(s * R_BASE + i * ZR, ZR)])
        return carry

    n_zero = R_BASE // ZR + 2 * (s == NS - 1).astype(jnp.int32)
    lax.fori_loop(0, n_zero, zloop, 0)
    plsc.subcore_barrier()

    ebase = wid * CH * C

    def chunk(k, carry):
        base = ebase + k * C
        pltpu.sync_copy(cols_hbm.at[pl.ds(base, C)], cols_v)
        pltpu.sync_copy(vals_hbm.at[pl.ds(base, C)], vals_v)
        pltpu.async_copy(x_hbm.at[cols_v], gath_v, semg).wait()

        @pl.when(k > 0)
        def _drain():
            pltpu.make_async_copy(sb0, acc.at[pl.ds(0, C)], sems0).wait()

        def sgroup(g, inner):
            vv = vals_v[pl.ds(g * 16, 16)]
            for l in range(16):
                v = vv[l]
                e = g * 16 + l
                for j in range(D // 16):
                    sl = pl.ds(j * 16, 16)
                    sb0[e, sl] = gath_v[e, sl] * v
            return inner

        lax.fori_loop(0, C // 16, sgroup, 0)

        pltpu.sync_copy(rows_hbm.at[pl.ds(base, C)], rows_v)
        pltpu.async_copy(sb0, acc.at[rows_v], sems0, add=True)
        return carry

    lax.fori_loop(0, CH, chunk, 0)
    pltpu.make_async_copy(sb0, acc.at[pl.ds(0, C)], sems0).wait()
    plsc.subcore_barrier()

    pltpu.sync_copy(acc.at[pl.ds(s * R_BASE, R_BASE)],
                    out_hbm.at[c, pl.ds(s * R_BASE, R_BASE)])

    @pl.when(s == NS - 1)
    def _tail_out():
        t = NS * R_BASE
        pltpu.sync_copy(acc.at[pl.ds(t, N_NODES - t)],
                        out_hbm.at[c, pl.ds(t, N_NODES - t)])


def _gcn_sc_partials(rows, cols, vals, x):
    mesh = plsc.VectorSubcoreMesh(core_axis_name="c", subcore_axis_name="s")
    kfn = pl.kernel(
        _sc_body,
        out_type=jax.ShapeDtypeStruct((NC, N_NODES, D), jnp.float32),
        mesh=mesh,
        scratch_types=[
            pltpu.VMEM((C,), jnp.int32),
            pltpu.VMEM((C,), jnp.int32),
            pltpu.VMEM((C,), jnp.float32),
            pltpu.VMEM((C, D), jnp.float32),
            pltpu.VMEM((C, D), jnp.float32),
            pltpu.VMEM_SHARED((N_NODES, D), jnp.float32),
            pltpu.SemaphoreType.DMA,
            pltpu.SemaphoreType.DMA,
        ],
    )
    return kfn(rows, cols, vals, x)


def _tc_body(p_ref, x_ref, wp_ref, ws_ref, b_ref, o_ref):
    agg = p_ref[0] + p_ref[1]
    o_ref[...] = (
        jnp.dot(agg, wp_ref[...], preferred_element_type=jnp.float32)
        + jnp.dot(x_ref[...], ws_ref[...], preferred_element_type=jnp.float32)
        + b_ref[...]
    )


def _gcn_tc_combine(p, x, wp_t, ws_t, b):
    BR = 1000
    return pl.pallas_call(
        _tc_body,
        grid=(N_NODES // BR,),
        in_specs=[
            pl.BlockSpec((NC, BR, D), lambda i: (0, i, 0)),
            pl.BlockSpec((BR, D), lambda i: (i, 0)),
            pl.BlockSpec((D, D), lambda i: (0, 0)),
            pl.BlockSpec((D, D), lambda i: (0, 0)),
            pl.BlockSpec((1, D), lambda i: (0, 0)),
        ],
        out_specs=pl.BlockSpec((BR, D), lambda i: (i, 0)),
        out_shape=jax.ShapeDtypeStruct((N_NODES, D), jnp.float32),
    )(p, x, wp_t, ws_t, b)


@jax.jit
def _impl(edge_index, edge_values, X, W_pass, b_pass, W_self, b_self):
    rows = edge_index[0].astype(jnp.int32)
    cols = edge_index[1].astype(jnp.int32)
    vals = edge_values.astype(jnp.float32)
    pad = PAD_E - N_EDGES
    rows_p = jnp.pad(rows, (0, pad))
    cols_p = jnp.pad(cols, (0, pad))
    vals_p = jnp.pad(vals, (0, pad))
    p = _gcn_sc_partials(rows_p, cols_p, vals_p, X)
    b = (b_pass + b_self).reshape(1, D)
    return _gcn_tc_combine(p, X, W_pass.T, W_self.T, b)


def kernel(edge_index, edge_values, X, W_pass, b_pass, W_self, b_self):
    return _impl(edge_index, edge_values, X, W_pass, b_pass, W_self, b_self)


# speedup vs baseline: 1.0716x; 1.0716x over previous
import functools

import numpy as np
import jax
import jax.numpy as jnp
from jax import lax
from jax.experimental import pallas as pl
from jax.experimental.pallas import tpu as pltpu
from jax.experimental.pallas import tpu_sc as plsc

N_NODES = 10000
N_EDGES = 320000
D = 128

NC = 2
NS = 16
NW = NC * NS

C = 128
CH = 80
PAD_E = NW * CH * C

R_BASE = 624
ZR = 8


def _sc_body(rows_hbm, cv_hbm, x_hbm, out_hbm,
             cv_v, idx_v, rows_v, gath_v, sb0, acc, semg):
    c = lax.axis_index("c")
    s = lax.axis_index("s")
    wid = s * NC + c

    zeros16 = jnp.zeros((16,), jnp.float32)
    for r in range(ZR):
        for j in range(D // 16):
            sb0[r, pl.ds(j * 16, 16)] = zeros16

    def zloop(i, carry):
        pltpu.sync_copy(sb0, acc.at[pl.ds(s * R_BASE + i * ZR, ZR)])
        return carry

    n_zero = R_BASE // ZR + 2 * (s == NS - 1).astype(jnp.int32)
    lax.fori_loop(0, n_zero, zloop, 0)
    plsc.subcore_barrier()

    ebase = wid * CH * C

    def chunk(k, carry):
        base = ebase + k * C
        pltpu.sync_copy(cv_hbm.at[pl.ds(2 * base, 2 * C)], cv_v)
        for t in range(C // 16):
            idx_v[pl.ds(t * 16, 16)] = cv_v[pl.ds(t * 16, 16)].astype(
                jnp.int32)
        pltpu.async_copy(x_hbm.at[idx_v], gath_v, semg).wait()

        def sgroup(g, inner):
            vv = cv_v[pl.ds(C + g * 16, 16)]
            for l in range(16):
                v = vv[l]
                e = g * 16 + l
                for j in range(D // 16):
                    sl = pl.ds(j * 16, 16)
                    gath_v[e, sl] = gath_v[e, sl] * v
            return inner

        lax.fori_loop(0, C // 16, sgroup, 0)

        pltpu.sync_copy(rows_hbm.at[pl.ds(base, C)], rows_v)
        pltpu.sync_copy(gath_v, acc.at[rows_v], add=True)
        return carry

    lax.fori_loop(0, CH, chunk, 0)
    plsc.subcore_barrier()

    pltpu.sync_copy(acc.at[pl.ds(s * R_BASE, R_BASE)],
                    out_hbm.at[c, pl.ds(s * R_BASE, R_BASE)])

    @pl.when(s == NS - 1)
    def _tail_out():
        t = NS * R_BASE
        pltpu.sync_copy(acc.at[pl.ds(t, N_NODES - t)],
                        out_hbm.at[c, pl.ds(t, N_NODES - t)])


def _gcn_sc_partials(rows, cv, x):
    mesh = plsc.VectorSubcoreMesh(core_axis_name="c", subcore_axis_name="s")
    kfn = pl.kernel(
        _sc_body,
        out_type=jax.ShapeDtypeStruct((NC, N_NODES, D), jnp.float32),
        mesh=mesh,
        scratch_types=[
            pltpu.VMEM((2 * C,), jnp.float32),
            pltpu.VMEM((C,), jnp.int32),
            pltpu.VMEM((C,), jnp.int32),
            pltpu.VMEM((C, D), jnp.float32),
            pltpu.VMEM((ZR, D), jnp.float32),
            pltpu.VMEM_SHARED((N_NODES, D), jnp.float32),
            pltpu.SemaphoreType.DMA,
        ],
    )
    return kfn(rows, cv, x)


def _tc_body(p_ref, x_ref, wp_ref, ws_ref, b_ref, o_ref):
    agg = p_ref[0] + p_ref[1]
    o_ref[...] = (
        jnp.dot(agg, wp_ref[...], preferred_element_type=jnp.float32)
        + jnp.dot(x_ref[...], ws_ref[...], preferred_element_type=jnp.float32)
        + b_ref[...]
    )


def _gcn_tc_combine(p, x, wp_t, ws_t, b):
    BR = 1000
    return pl.pallas_call(
        _tc_body,
        grid=(N_NODES // BR,),
        in_specs=[
            pl.BlockSpec((NC, BR, D), lambda i: (0, i, 0)),
            pl.BlockSpec((BR, D), lambda i: (i, 0)),
            pl.BlockSpec((D, D), lambda i: (0, 0)),
            pl.BlockSpec((D, D), lambda i: (0, 0)),
            pl.BlockSpec((1, D), lambda i: (0, 0)),
        ],
        out_specs=pl.BlockSpec((BR, D), lambda i: (i, 0)),
        out_shape=jax.ShapeDtypeStruct((N_NODES, D), jnp.float32),
    )(p, x, wp_t, ws_t, b)


@jax.jit
def _impl(edge_index, edge_values, X, W_pass, b_pass, W_self, b_self):
    rows = edge_index[0].astype(jnp.int32)
    cols = edge_index[1].astype(jnp.int32)
    vals = edge_values.astype(jnp.float32)
    pad = PAD_E - N_EDGES
    rows_p = jnp.pad(rows, (0, pad))
    cols_p = jnp.pad(cols, (0, pad))
    vals_p = jnp.pad(vals, (0, pad))
    cv = jnp.concatenate(
        [cols_p.astype(jnp.float32).reshape(-1, 1, C),
         vals_p.reshape(-1, 1, C)],
        axis=1).reshape(-1)
    p = _gcn_sc_partials(rows_p, cv, X)
    b = (b_pass + b_self).reshape(1, D)
    return _gcn_tc_combine(p, X, W_pass.T, W_self.T, b)


def kernel(edge_index, edge_values, X, W_pass, b_pass, W_self, b_self):
    return _impl(edge_index, edge_values, X, W_pass, b_pass, W_self, b_self)


# speedup vs baseline: 1.9933x; 1.8601x over previous
import functools

import jax
import jax.numpy as jnp
from jax import lax
from jax.experimental import pallas as pl
from jax.experimental.pallas import tpu as pltpu
from jax.experimental.pallas import tpu_sc as plsc

N_NODES = 10000
N_EDGES = 320000
D = 128

NC = 2
NS = 16
NW = NC * NS

C = 128
CHUNKS = N_EDGES // C
FULL_ROUNDS = CHUNKS // NW
TAIL = CHUNKS - FULL_ROUNDS * NW

R_BASE = 624
ZR = 16


def _sc_body(rows_hbm, cols_hbm, vals_hbm, x_hbm, out_hbm,
             cols_v, rows_v, vals_v, gath_v, zero_v, acc, sem):
    c = lax.axis_index("c")
    s = lax.axis_index("s")
    wid = s * NC + c

    zeros16 = jnp.zeros((16,), jnp.float32)
    for r in range(ZR):
        for j in range(D // 16):
            zero_v[r, pl.ds(j * 16, 16)] = zeros16

    def zloop(i, carry):
        pltpu.sync_copy(zero_v, acc.at[pl.ds(s * R_BASE + i * ZR, ZR)])
        return carry

    n_zero = R_BASE // ZR + (s == NS - 1).astype(jnp.int32)
    lax.fori_loop(0, n_zero, zloop, 0)
    plsc.subcore_barrier()

    def echunk(k, carry):
        base = (wid + k * NW) * C
        pltpu.sync_copy(cols_hbm.at[pl.ds(base, C)], cols_v)
        pltpu.sync_copy(rows_hbm.at[pl.ds(base, C)], rows_v)
        pltpu.sync_copy(vals_hbm.at[pl.ds(base, C)], vals_v)
        pltpu.async_copy(x_hbm.at[cols_v], gath_v, sem).wait()

        def scale(g, inner):
            vv = vals_v[pl.ds(g * 16, 16)]
            for l in range(16):
                v = vv[l]
                e = g * 16 + l
                for j in range(D // 16):
                    sl = pl.ds(j * 16, 16)
                    gath_v[e, sl] = gath_v[e, sl] * v
            return inner

        lax.fori_loop(0, C // 16, scale, 0)
        pltpu.sync_copy(gath_v, acc.at[rows_v], add=True)
        return carry

    nch = FULL_ROUNDS + (wid < TAIL).astype(jnp.int32)
    lax.fori_loop(0, nch, echunk, 0)
    plsc.subcore_barrier()

    pltpu.sync_copy(acc.at[pl.ds(s * R_BASE, R_BASE)],
                    out_hbm.at[c, pl.ds(s * R_BASE, R_BASE)])

    @pl.when(s == NS - 1)
    def _tail_out():
        t = NS * R_BASE
        pltpu.sync_copy(acc.at[pl.ds(t, N_NODES - t)],
                        out_hbm.at[c, pl.ds(t, N_NODES - t)])


def _gcn_sc_partials(rows, cols, vals, x):
    mesh = plsc.VectorSubcoreMesh(core_axis_name="c", subcore_axis_name="s")
    kfn = pl.kernel(
        _sc_body,
        out_type=jax.ShapeDtypeStruct((NC, N_NODES, D), jnp.float32),
        mesh=mesh,
        scratch_types=[
            pltpu.VMEM((C,), jnp.int32),
            pltpu.VMEM((C,), jnp.int32),
            pltpu.VMEM((C,), jnp.float32),
            pltpu.VMEM((C, D), jnp.float32),
            pltpu.VMEM((ZR, D), jnp.float32),
            pltpu.VMEM_SHARED((N_NODES, D), jnp.float32),
            pltpu.SemaphoreType.DMA,
        ],
    )
    return kfn(rows, cols, vals, x)


def _tc_body(p_ref, x_ref, wp_ref, ws_ref, b_ref, o_ref):
    agg = p_ref[0] + p_ref[1]
    o_ref[...] = (
        jnp.dot(agg, wp_ref[...], preferred_element_type=jnp.float32)
        + jnp.dot(x_ref[...], ws_ref[...], preferred_element_type=jnp.float32)
        + b_ref[...]
    )


def _gcn_tc_combine(p, x, wp_t, ws_t, b):
    BR = 1000
    return pl.pallas_call(
        _tc_body,
        grid=(N_NODES // BR,),
        in_specs=[
            pl.BlockSpec((NC, BR, D), lambda i: (0, i, 0)),
            pl.BlockSpec((BR, D), lambda i: (i, 0)),
            pl.BlockSpec((D, D), lambda i: (0, 0)),
            pl.BlockSpec((D, D), lambda i: (0, 0)),
            pl.BlockSpec((1, D), lambda i: (0, 0)),
        ],
        out_specs=pl.BlockSpec((BR, D), lambda i: (i, 0)),
        out_shape=jax.ShapeDtypeStruct((N_NODES, D), jnp.float32),
    )(p, x, wp_t, ws_t, b)


@jax.jit
def _impl(edge_index, edge_values, X, W_pass, b_pass, W_self, b_self):
    rows = edge_index[0].astype(jnp.int32)
    cols = edge_index[1].astype(jnp.int32)
    p = _gcn_sc_partials(rows, cols, edge_values, X)
    b = (b_pass + b_self).reshape(1, D)
    return _gcn_tc_combine(p, X, W_pass.T, W_self.T, b)


def kernel(edge_index, edge_values, X, W_pass, b_pass, W_self, b_self):
    return _impl(edge_index, edge_values, X, W_pass, b_pass, W_self, b_self)


# speedup vs baseline: 1.9939x; 1.0003x over previous
import functools

import jax
import jax.numpy as jnp
from jax import lax
from jax.experimental import pallas as pl
from jax.experimental.pallas import tpu as pltpu
from jax.experimental.pallas import tpu_sc as plsc

N_NODES = 10000
N_EDGES = 320000
D = 128

NC = 2
NS = 16
NW = NC * NS

C = 128
CHUNKS = N_EDGES // C
FULL_ROUNDS = CHUNKS // NW
TAIL = CHUNKS - FULL_ROUNDS * NW

R_BASE = 624
ZR = 80


def _sc_body(rows_hbm, cols_hbm, vals_hbm, x_hbm, out_hbm,
             cols_v, rows_v, vals_v, gath_v, zero_v, acc, sem):
    c = lax.axis_index("c")
    s = lax.axis_index("s")
    wid = s * NC + c

    zeros16 = jnp.zeros((16,), jnp.float32)
    for r in range(ZR):
        for j in range(D // 16):
            zero_v[r, pl.ds(j * 16, 16)] = zeros16

    def zloop(i, carry):
        pltpu.sync_copy(zero_v, acc.at[pl.ds(s * 640 + i * ZR, ZR)])
        return carry

    n_zero = 8 - 3 * (s == NS - 1).astype(jnp.int32)
    lax.fori_loop(0, n_zero, zloop, 0)
    plsc.subcore_barrier()

    def echunk(k, carry):
        base = (wid + k * NW) * C
        pltpu.sync_copy(cols_hbm.at[pl.ds(base, C)], cols_v)
        pltpu.sync_copy(rows_hbm.at[pl.ds(base, C)], rows_v)
        pltpu.sync_copy(vals_hbm.at[pl.ds(base, C)], vals_v)
        pltpu.async_copy(x_hbm.at[cols_v], gath_v, sem).wait()

        def scale(g, inner):
            vv = vals_v[pl.ds(g * 16, 16)]
            for l in range(16):
                v = vv[l]
                e = g * 16 + l
                for j in range(D // 16):
                    sl = pl.ds(j * 16, 16)
                    gath_v[e, sl] = gath_v[e, sl] * v
            return inner

        lax.fori_loop(0, C // 16, scale, 0)
        pltpu.sync_copy(gath_v, acc.at[rows_v], add=True)
        return carry

    nch = FULL_ROUNDS + (wid < TAIL).astype(jnp.int32)
    lax.fori_loop(0, nch, echunk, 0)
    plsc.subcore_barrier()

    pltpu.sync_copy(acc.at[pl.ds(s * R_BASE, R_BASE)],
                    out_hbm.at[c, pl.ds(s * R_BASE, R_BASE)])

    @pl.when(s == NS - 1)
    def _tail_out():
        t = NS * R_BASE
        pltpu.sync_copy(acc.at[pl.ds(t, N_NODES - t)],
                        out_hbm.at[c, pl.ds(t, N_NODES - t)])


def _gcn_sc_partials(rows, cols, vals, x):
    mesh = plsc.VectorSubcoreMesh(core_axis_name="c", subcore_axis_name="s")
    kfn = pl.kernel(
        _sc_body,
        out_type=jax.ShapeDtypeStruct((NC, N_NODES, D), jnp.float32),
        mesh=mesh,
        scratch_types=[
            pltpu.VMEM((C,), jnp.int32),
            pltpu.VMEM((C,), jnp.int32),
            pltpu.VMEM((C,), jnp.float32),
            pltpu.VMEM((C, D), jnp.float32),
            pltpu.VMEM((ZR, D), jnp.float32),
            pltpu.VMEM_SHARED((N_NODES, D), jnp.float32),
            pltpu.SemaphoreType.DMA,
        ],
    )
    return kfn(rows, cols, vals, x)


def _tc_body(p_ref, x_ref, wp_ref, ws_ref, b_ref, o_ref):
    agg = p_ref[0] + p_ref[1]
    o_ref[...] = (
        jnp.dot(agg, wp_ref[...], preferred_element_type=jnp.float32)
        + jnp.dot(x_ref[...], ws_ref[...], preferred_element_type=jnp.float32)
        + b_ref[...]
    )


def _gcn_tc_combine(p, x, wp_t, ws_t, b):
    BR = 1000
    return pl.pallas_call(
        _tc_body,
        grid=(N_NODES // BR,),
        in_specs=[
            pl.BlockSpec((NC, BR, D), lambda i: (0, i, 0)),
            pl.BlockSpec((BR, D), lambda i: (i, 0)),
            pl.BlockSpec((D, D), lambda i: (0, 0)),
            pl.BlockSpec((D, D), lambda i: (0, 0)),
            pl.BlockSpec((1, D), lambda i: (0, 0)),
        ],
        out_specs=pl.BlockSpec((BR, D), lambda i: (i, 0)),
        out_shape=jax.ShapeDtypeStruct((N_NODES, D), jnp.float32),
    )(p, x, wp_t, ws_t, b)


@jax.jit
def _impl(edge_index, edge_values, X, W_pass, b_pass, W_self, b_self):
    rows = edge_index[0].astype(jnp.int32)
    cols = edge_index[1].astype(jnp.int32)
    p = _gcn_sc_partials(rows, cols, edge_values, X)
    b = (b_pass + b_self).reshape(1, D)
    return _gcn_tc_combine(p, X, W_pass.T, W_self.T, b)


def kernel(edge_index, edge_values, X, W_pass, b_pass, W_self, b_self):
    return _impl(edge_index, edge_values, X, W_pass, b_pass, W_self, b_self)
